# Initial kernel scaffold; baseline (speedup 1.0000x reference)
import jax
import jax.numpy as jnp
from jax.experimental import pallas as pl


def kernel(features, edge_index, W1, W2, W3):
    raise NotImplementedError("write your pallas kernel here")


# speedup vs baseline: 6.4351x; 6.4351x over previous
import functools

import jax
import jax.numpy as jnp
from jax import lax
from jax.experimental import pallas as pl
from jax.experimental.pallas import tpu as pltpu
from jax.experimental.pallas import tpu_sc as plsc

N = 10000
E = 320000
D_IN = 128
D_HID = 64
D_LAT = 32

NC = 2
NS = 16
NW = NC * NS
EPW = E // NW
CHUNK = 80
NCHUNK = EPW // CHUNK
NP = 10240
RPS = NP // NS

_mesh = plsc.VectorSubcoreMesh(core_axis_name="c", subcore_axis_name="s")


_SC_PARAMS = pltpu.CompilerParams(use_tc_tiling_on_sc=False)


@functools.partial(
    pl.kernel,
    out_type=jax.ShapeDtypeStruct((NC * NP, 16), jnp.float32),
    mesh=_mesh,
    compiler_params=_SC_PARAMS,
    scratch_types=[
        pltpu.VMEM((NCHUNK, CHUNK), jnp.int32),
        pltpu.VMEM((CHUNK, 16), jnp.float32),
        pltpu.VMEM((RPS, 16), jnp.float32),
        pltpu.VMEM_SHARED((NP, 16), jnp.float32),
    ],
)
def _sc_degree(dst_hbm, out_hbm, dst_v, ones_v, zero_v, acc):
    c = lax.axis_index("c")
    s = lax.axis_index("s")
    wid = c * NS + s

    def zfill(i, carry):
        zero_v[i, :] = jnp.zeros((16,), jnp.float32)
        return carry

    lax.fori_loop(0, RPS, zfill, 0)

    def ofill(i, carry):
        ones_v[i, :] = jnp.full((16,), 1.0, jnp.float32)
        return carry

    lax.fori_loop(0, CHUNK, ofill, 0)

    pltpu.sync_copy(zero_v, acc.at[pl.ds(s * RPS, RPS)])
    pltpu.sync_copy(dst_hbm.at[wid], dst_v)
    plsc.subcore_barrier()

    def step(k, carry):
        pltpu.sync_copy(ones_v, acc.at[dst_v.at[k]], add=True)
        return carry

    lax.fori_loop(0, NCHUNK, step, 0)

    plsc.subcore_barrier()
    pltpu.sync_copy(acc.at[pl.ds(s * RPS, RPS)],
                    out_hbm.at[pl.ds(c * NP + s * RPS, RPS)])


@functools.partial(
    pl.kernel,
    out_type=jax.ShapeDtypeStruct((NC * NP, D_HID), jnp.float32),
    mesh=_mesh,
    compiler_params=_SC_PARAMS,
    scratch_types=[
        pltpu.VMEM((NCHUNK, CHUNK), jnp.int32),
        pltpu.VMEM((NCHUNK, CHUNK), jnp.int32),
        pltpu.VMEM((CHUNK, D_HID), jnp.float32),
        pltpu.VMEM((RPS, D_HID), jnp.float32),
        pltpu.VMEM_SHARED((NP, D_HID), jnp.float32),
        pltpu.SemaphoreType.DMA,
    ],
)
def _sc_aggregate(feat_hbm, src_hbm, dst_hbm, out_hbm,
                  src_v, dst_v, rows_v, zero_v, acc, sem):
    c = lax.axis_index("c")
    s = lax.axis_index("s")
    wid = c * NS + s

    def zfill(i, carry):
        for j in range(D_HID // 16):
            zero_v[i, pl.ds(j * 16, 16)] = jnp.zeros((16,), jnp.float32)
        return carry

    lax.fori_loop(0, RPS, zfill, 0)

    pltpu.sync_copy(zero_v, acc.at[pl.ds(s * RPS, RPS)])
    pltpu.sync_copy(src_hbm.at[wid], src_v)
    pltpu.sync_copy(dst_hbm.at[wid], dst_v)
    plsc.subcore_barrier()

    def step(k, carry):
        pltpu.async_copy(feat_hbm.at[src_v.at[k]], rows_v, sem).wait()
        pltpu.sync_copy(rows_v, acc.at[dst_v.at[k]], add=True)
        return carry

    lax.fori_loop(0, NCHUNK, step, 0)

    plsc.subcore_barrier()
    pltpu.sync_copy(acc.at[pl.ds(s * RPS, RPS)],
                    out_hbm.at[pl.ds(c * NP + s * RPS, RPS)])


BM = 1000


def _dinv_block(degp_ref):
    deg = degp_ref[0, :, 0:1] + degp_ref[1, :, 0:1] + 1.0
    return lax.rsqrt(deg)


def _tc_embed(x_ref, w1_ref, degp_ref, hn_ref):
    dinv = _dinv_block(degp_ref)
    h1 = jnp.dot(x_ref[:], w1_ref[:], preferred_element_type=jnp.float32)
    hn_ref[:] = h1 * dinv


def _tc_mid(p_ref, hn_ref, degp_ref, out_ref):
    dinv = _dinv_block(degp_ref)
    agg = p_ref[0] + p_ref[1] + hn_ref[:]
    hidden = jnp.maximum(agg * dinv, 0.0)
    out_ref[:] = hidden * dinv


def _tc_latent(p_ref, hn2_ref, degp_ref, w2_ref, w3_ref, mu_ref, lv_ref):
    dinv = _dinv_block(degp_ref)
    sagg = (p_ref[0] + p_ref[1] + hn2_ref[:]) * dinv
    mu_ref[:] = jnp.dot(sagg, w2_ref[:], preferred_element_type=jnp.float32)
    lv_ref[:] = jnp.dot(sagg, w3_ref[:], preferred_element_type=jnp.float32)


BMD = 512
BND = 2048


def _tc_decoder(zi_ref, zj_ref, out_ref):
    prod = lax.dot_general(zi_ref[:], zj_ref[:], (((1,), (1,)), ((), ())),
                           preferred_element_type=jnp.float32)
    out_ref[:] = jax.nn.sigmoid(prod)


def kernel(features, edge_index, W1, W2, W3):
    src2 = edge_index[0].reshape(NW, NCHUNK, CHUNK)
    dst2 = edge_index[1].reshape(NW, NCHUNK, CHUNK)

    degp = _sc_degree(dst2).reshape(NC, NP, 16)

    hn = pl.pallas_call(
        _tc_embed,
        grid=(N // BM,),
        in_specs=[
            pl.BlockSpec((BM, D_IN), lambda i: (i, 0)),
            pl.BlockSpec((D_IN, D_HID), lambda i: (0, 0)),
            pl.BlockSpec((NC, BM, 16), lambda i: (0, i, 0)),
        ],
        out_specs=pl.BlockSpec((BM, D_HID), lambda i: (i, 0)),
        out_shape=jax.ShapeDtypeStruct((N, D_HID), jnp.float32),
    )(features, W1, degp)

    p1 = _sc_aggregate(hn, src2, dst2).reshape(NC, NP, D_HID)

    hn2 = pl.pallas_call(
        _tc_mid,
        grid=(N // BM,),
        in_specs=[
            pl.BlockSpec((NC, BM, D_HID), lambda i: (0, i, 0)),
            pl.BlockSpec((BM, D_HID), lambda i: (i, 0)),
            pl.BlockSpec((NC, BM, 16), lambda i: (0, i, 0)),
        ],
        out_specs=pl.BlockSpec((BM, D_HID), lambda i: (i, 0)),
        out_shape=jax.ShapeDtypeStruct((N, D_HID), jnp.float32),
    )(p1, hn, degp)

    p2 = _sc_aggregate(hn2, src2, dst2).reshape(NC, NP, D_HID)

    mu, logvar = pl.pallas_call(
        _tc_latent,
        grid=(N // BM,),
        in_specs=[
            pl.BlockSpec((NC, BM, D_HID), lambda i: (0, i, 0)),
            pl.BlockSpec((BM, D_HID), lambda i: (i, 0)),
            pl.BlockSpec((NC, BM, 16), lambda i: (0, i, 0)),
            pl.BlockSpec((D_HID, D_LAT), lambda i: (0, 0)),
            pl.BlockSpec((D_HID, D_LAT), lambda i: (0, 0)),
        ],
        out_specs=[
            pl.BlockSpec((BM, D_LAT), lambda i: (i, 0)),
            pl.BlockSpec((BM, D_LAT), lambda i: (i, 0)),
        ],
        out_shape=[
            jax.ShapeDtypeStruct((N, D_LAT), jnp.float32),
            jax.ShapeDtypeStruct((N, D_LAT), jnp.float32),
        ],
    )(p2, hn2, degp, W2, W3)

    adj = pl.pallas_call(
        _tc_decoder,
        grid=(pl.cdiv(N, BMD), pl.cdiv(N, BND)),
        in_specs=[
            pl.BlockSpec((BMD, D_LAT), lambda i, j: (i, 0)),
            pl.BlockSpec((BND, D_LAT), lambda i, j: (j, 0)),
        ],
        out_specs=pl.BlockSpec((BMD, BND), lambda i, j: (i, j)),
        out_shape=jax.ShapeDtypeStruct((N, N), jnp.float32),
    )(mu, mu)

    return adj, mu, logvar


# speedup vs baseline: 8.8913x; 1.3817x over previous
import functools

import jax
import jax.numpy as jnp
from jax import lax
from jax.experimental import pallas as pl
from jax.experimental.pallas import tpu as pltpu
from jax.experimental.pallas import tpu_sc as plsc

N = 10000
E = 320000
D_IN = 128
D_HID = 64
D_LAT = 32

NC = 2
NS = 16
NW = NC * NS
EPW = E // NW
CHUNK = 125
NCHUNK = EPW // CHUNK
RING = 5
NGROUP = NCHUNK // RING
NP = 10240
RPS = NP // NS

_mesh = plsc.VectorSubcoreMesh(core_axis_name="c", subcore_axis_name="s")


_SC_PARAMS = pltpu.CompilerParams(use_tc_tiling_on_sc=False)


@functools.partial(
    pl.kernel,
    out_type=jax.ShapeDtypeStruct((NC * NP, 16), jnp.float32),
    mesh=_mesh,
    compiler_params=_SC_PARAMS,
    scratch_types=[
        pltpu.VMEM((NCHUNK, CHUNK), jnp.int32),
        pltpu.VMEM((CHUNK, 16), jnp.float32),
        pltpu.VMEM((RPS, 16), jnp.float32),
        pltpu.VMEM_SHARED((NP, 16), jnp.float32),
    ],
)
def _sc_degree(dst_hbm, out_hbm, dst_v, ones_v, zero_v, acc):
    c = lax.axis_index("c")
    s = lax.axis_index("s")
    wid = c * NS + s

    def zfill(i, carry):
        zero_v[i, :] = jnp.zeros((16,), jnp.float32)
        return carry

    lax.fori_loop(0, RPS, zfill, 0)

    def ofill(i, carry):
        ones_v[i, :] = jnp.full((16,), 1.0, jnp.float32)
        return carry

    lax.fori_loop(0, CHUNK, ofill, 0)

    pltpu.sync_copy(zero_v, acc.at[pl.ds(s * RPS, RPS)])
    pltpu.sync_copy(dst_hbm.at[wid], dst_v)
    plsc.subcore_barrier()

    def step(k, carry):
        pltpu.sync_copy(ones_v, acc.at[dst_v.at[k]], add=True)
        return carry

    lax.fori_loop(0, NCHUNK, step, 0)

    plsc.subcore_barrier()
    pltpu.sync_copy(acc.at[pl.ds(s * RPS, RPS)],
                    out_hbm.at[pl.ds(c * NP + s * RPS, RPS)])


@functools.partial(
    pl.kernel,
    out_type=jax.ShapeDtypeStruct((NC * NP, D_HID), jnp.float32),
    mesh=_mesh,
    compiler_params=_SC_PARAMS,
    scratch_types=[
        pltpu.VMEM((NCHUNK, CHUNK), jnp.int32),
        pltpu.VMEM((NCHUNK, CHUNK), jnp.int32),
        pltpu.VMEM((RING, CHUNK, D_HID), jnp.float32),
        pltpu.VMEM((RPS // 4, D_HID), jnp.float32),
        pltpu.VMEM_SHARED((NP, D_HID), jnp.float32),
    ] + [pltpu.SemaphoreType.DMA] * (2 * RING),
)
def _sc_aggregate(feat_hbm, src_hbm, dst_hbm, out_hbm,
                  src_v, dst_v, rows_v, zero_v, acc, *sems):
    gsem = sems[:RING]
    ssem = sems[RING:]
    c = lax.axis_index("c")
    s = lax.axis_index("s")
    wid = c * NS + s

    def zfill(i, carry):
        for j in range(D_HID // 16):
            zero_v[i, pl.ds(j * 16, 16)] = jnp.zeros((16,), jnp.float32)
        return carry

    lax.fori_loop(0, RPS // 4, zfill, 0)

    for q in range(4):
        pltpu.sync_copy(zero_v, acc.at[pl.ds(s * RPS + q * (RPS // 4), RPS // 4)])
    pltpu.sync_copy(src_hbm.at[wid], src_v)
    pltpu.sync_copy(dst_hbm.at[wid], dst_v)
    plsc.subcore_barrier()

    for b in range(RING):
        pltpu.async_copy(feat_hbm.at[src_v.at[b]], rows_v.at[b], gsem[b])

    def group(o, carry):
        descs = []
        for b in range(RING):
            k = o * RING + b
            pltpu.make_async_copy(feat_hbm.at[src_v.at[k]],
                                  rows_v.at[b], gsem[b]).wait()
            descs.append(pltpu.async_copy(rows_v.at[b],
                                          acc.at[dst_v.at[k]], ssem[b],
                                          add=True))

        @pl.when(o < NGROUP - 1)
        def _refill():
            for b in range(RING):
                descs[b].wait()
                k2 = (o + 1) * RING + b
                pltpu.async_copy(feat_hbm.at[src_v.at[k2]],
                                 rows_v.at[b], gsem[b])

        return carry

    lax.fori_loop(0, NGROUP, group, 0)

    for b in range(RING):
        kf = (NGROUP - 1) * RING + b
        pltpu.make_async_copy(rows_v.at[b], acc.at[dst_v.at[kf]],
                              ssem[b]).wait()

    plsc.subcore_barrier()
    pltpu.sync_copy(acc.at[pl.ds(s * RPS, RPS)],
                    out_hbm.at[pl.ds(c * NP + s * RPS, RPS)])


BM = 1000


def _dinv_block(degp_ref):
    deg = degp_ref[0, :, 0:1] + degp_ref[1, :, 0:1] + 1.0
    return lax.rsqrt(deg)


def _tc_embed(x_ref, w1_ref, degp_ref, hn_ref):
    dinv = _dinv_block(degp_ref)
    h1 = jnp.dot(x_ref[:], w1_ref[:], preferred_element_type=jnp.float32)
    hn_ref[:] = h1 * dinv


def _tc_mid(p_ref, hn_ref, degp_ref, out_ref):
    dinv = _dinv_block(degp_ref)
    agg = p_ref[0] + p_ref[1] + hn_ref[:]
    hidden = jnp.maximum(agg * dinv, 0.0)
    out_ref[:] = hidden * dinv


def _tc_latent(p_ref, hn2_ref, degp_ref, w2_ref, w3_ref, mu_ref, lv_ref):
    dinv = _dinv_block(degp_ref)
    sagg = (p_ref[0] + p_ref[1] + hn2_ref[:]) * dinv
    mu_ref[:] = jnp.dot(sagg, w2_ref[:], preferred_element_type=jnp.float32)
    lv_ref[:] = jnp.dot(sagg, w3_ref[:], preferred_element_type=jnp.float32)


BMD = 512
BND = 2048


def _tc_decoder(zi_ref, zj_ref, out_ref):
    prod = lax.dot_general(zi_ref[:], zj_ref[:], (((1,), (1,)), ((), ())),
                           preferred_element_type=jnp.float32)
    out_ref[:] = jax.nn.sigmoid(prod)


def kernel(features, edge_index, W1, W2, W3):
    src2 = edge_index[0].reshape(NW, NCHUNK, CHUNK)
    dst2 = edge_index[1].reshape(NW, NCHUNK, CHUNK)

    degp = _sc_degree(dst2).reshape(NC, NP, 16)

    hn = pl.pallas_call(
        _tc_embed,
        grid=(N // BM,),
        in_specs=[
            pl.BlockSpec((BM, D_IN), lambda i: (i, 0)),
            pl.BlockSpec((D_IN, D_HID), lambda i: (0, 0)),
            pl.BlockSpec((NC, BM, 16), lambda i: (0, i, 0)),
        ],
        out_specs=pl.BlockSpec((BM, D_HID), lambda i: (i, 0)),
        out_shape=jax.ShapeDtypeStruct((N, D_HID), jnp.float32),
    )(features, W1, degp)

    p1 = _sc_aggregate(hn, src2, dst2).reshape(NC, NP, D_HID)

    hn2 = pl.pallas_call(
        _tc_mid,
        grid=(N // BM,),
        in_specs=[
            pl.BlockSpec((NC, BM, D_HID), lambda i: (0, i, 0)),
            pl.BlockSpec((BM, D_HID), lambda i: (i, 0)),
            pl.BlockSpec((NC, BM, 16), lambda i: (0, i, 0)),
        ],
        out_specs=pl.BlockSpec((BM, D_HID), lambda i: (i, 0)),
        out_shape=jax.ShapeDtypeStruct((N, D_HID), jnp.float32),
    )(p1, hn, degp)

    p2 = _sc_aggregate(hn2, src2, dst2).reshape(NC, NP, D_HID)

    mu, logvar = pl.pallas_call(
        _tc_latent,
        grid=(N // BM,),
        in_specs=[
            pl.BlockSpec((NC, BM, D_HID), lambda i: (0, i, 0)),
            pl.BlockSpec((BM, D_HID), lambda i: (i, 0)),
            pl.BlockSpec((NC, BM, 16), lambda i: (0, i, 0)),
            pl.BlockSpec((D_HID, D_LAT), lambda i: (0, 0)),
            pl.BlockSpec((D_HID, D_LAT), lambda i: (0, 0)),
        ],
        out_specs=[
            pl.BlockSpec((BM, D_LAT), lambda i: (i, 0)),
            pl.BlockSpec((BM, D_LAT), lambda i: (i, 0)),
        ],
        out_shape=[
            jax.ShapeDtypeStruct((N, D_LAT), jnp.float32),
            jax.ShapeDtypeStruct((N, D_LAT), jnp.float32),
        ],
    )(p2, hn2, degp, W2, W3)

    adj = pl.pallas_call(
        _tc_decoder,
        grid=(pl.cdiv(N, BMD), pl.cdiv(N, BND)),
        in_specs=[
            pl.BlockSpec((BMD, D_LAT), lambda i, j: (i, 0)),
            pl.BlockSpec((BND, D_LAT), lambda i, j: (j, 0)),
        ],
        out_specs=pl.BlockSpec((BMD, BND), lambda i, j: (i, j)),
        out_shape=jax.ShapeDtypeStruct((N, N), jnp.float32),
    )(mu, mu)

    return adj, mu, logvar


# speedup vs baseline: 9.6421x; 1.0844x over previous
import functools

import jax
import jax.numpy as jnp
from jax import lax
from jax.experimental import pallas as pl
from jax.experimental.pallas import tpu as pltpu
from jax.experimental.pallas import tpu_sc as plsc

N = 10000
E = 320000
D_IN = 128
D_HID = 64
D_LAT = 32

NC = 2
NS = 16
NW = NC * NS
EPW = E // NW
CHUNK = 125
NCHUNK = EPW // CHUNK
RING = 5
NGROUP = NCHUNK // RING
NP = 10240
RPS = NP // NS

_mesh = plsc.VectorSubcoreMesh(core_axis_name="c", subcore_axis_name="s")


_SC_PARAMS = pltpu.CompilerParams(use_tc_tiling_on_sc=False)


@functools.partial(
    pl.kernel,
    out_type=jax.ShapeDtypeStruct((NC, NP, 16), jnp.float32),
    mesh=_mesh,
    compiler_params=_SC_PARAMS,
    scratch_types=[
        pltpu.VMEM((NCHUNK, CHUNK), jnp.int32),
        pltpu.VMEM((CHUNK, 16), jnp.float32),
        pltpu.VMEM((RPS, 16), jnp.float32),
        pltpu.VMEM_SHARED((NP, 16), jnp.float32),
        pltpu.SemaphoreType.DMA,
    ],
)
def _sc_degree(edges_hbm, out_hbm, dst_v, ones_v, zero_v, acc, sem):
    c = lax.axis_index("c")
    s = lax.axis_index("s")
    wid = c * NS + s

    def zfill(i, carry):
        zero_v[i, :] = jnp.zeros((16,), jnp.float32)
        return carry

    lax.fori_loop(0, RPS, zfill, 0)

    def ofill(i, carry):
        ones_v[i, :] = jnp.full((16,), 1.0, jnp.float32)
        return carry

    lax.fori_loop(0, CHUNK, ofill, 0)

    pltpu.sync_copy(zero_v, acc.at[pl.ds(s * RPS, RPS)])
    pltpu.sync_copy(edges_hbm.at[1, wid], dst_v)
    plsc.subcore_barrier()

    def group(o, carry):
        for b in range(RING):
            pltpu.async_copy(ones_v, acc.at[dst_v.at[o * RING + b]], sem,
                             add=True)
        for b in range(RING):
            pltpu.make_async_copy(ones_v, acc.at[dst_v.at[o * RING + b]],
                                  sem).wait()
        return carry

    lax.fori_loop(0, NCHUNK // RING, group, 0)

    plsc.subcore_barrier()
    pltpu.sync_copy(acc.at[pl.ds(s * RPS, RPS)],
                    out_hbm.at[c].at[pl.ds(s * RPS, RPS)])


@functools.partial(
    pl.kernel,
    out_type=jax.ShapeDtypeStruct((NC, NP, D_HID), jnp.float32),
    mesh=_mesh,
    compiler_params=_SC_PARAMS,
    scratch_types=[
        pltpu.VMEM((NCHUNK, CHUNK), jnp.int32),
        pltpu.VMEM((NCHUNK, CHUNK), jnp.int32),
        pltpu.VMEM((RING, CHUNK, D_HID), jnp.float32),
        pltpu.VMEM((RPS // 4, D_HID), jnp.float32),
        pltpu.VMEM_SHARED((NP, D_HID), jnp.float32),
    ] + [pltpu.SemaphoreType.DMA] * (2 * RING),
)
def _sc_aggregate(feat_hbm, edges_hbm, out_hbm,
                  src_v, dst_v, rows_v, zero_v, acc, *sems):
    gsem = sems[:RING]
    ssem = sems[RING:]
    c = lax.axis_index("c")
    s = lax.axis_index("s")
    wid = c * NS + s

    def zfill(i, carry):
        for j in range(D_HID // 16):
            zero_v[i, pl.ds(j * 16, 16)] = jnp.zeros((16,), jnp.float32)
        return carry

    lax.fori_loop(0, RPS // 4, zfill, 0)

    for q in range(4):
        pltpu.sync_copy(zero_v, acc.at[pl.ds(s * RPS + q * (RPS // 4), RPS // 4)])
    pltpu.sync_copy(edges_hbm.at[0, wid], src_v)
    pltpu.sync_copy(edges_hbm.at[1, wid], dst_v)
    plsc.subcore_barrier()

    for b in range(RING):
        pltpu.async_copy(feat_hbm.at[src_v.at[b]], rows_v.at[b], gsem[b])

    def group(o, carry):
        descs = []
        for b in range(RING):
            k = o * RING + b
            pltpu.make_async_copy(feat_hbm.at[src_v.at[k]],
                                  rows_v.at[b], gsem[b]).wait()
            descs.append(pltpu.async_copy(rows_v.at[b],
                                          acc.at[dst_v.at[k]], ssem[b],
                                          add=True))

        @pl.when(o < NGROUP - 1)
        def _refill():
            for b in range(RING):
                descs[b].wait()
                k2 = (o + 1) * RING + b
                pltpu.async_copy(feat_hbm.at[src_v.at[k2]],
                                 rows_v.at[b], gsem[b])

        return carry

    lax.fori_loop(0, NGROUP, group, 0)

    for b in range(RING):
        kf = (NGROUP - 1) * RING + b
        pltpu.make_async_copy(rows_v.at[b], acc.at[dst_v.at[kf]],
                              ssem[b]).wait()

    plsc.subcore_barrier()
    pltpu.sync_copy(acc.at[pl.ds(s * RPS, RPS)],
                    out_hbm.at[c].at[pl.ds(s * RPS, RPS)])


BM = 1000


def _dinv_block(degp_ref):
    deg = degp_ref[0, :, 0:1] + degp_ref[1, :, 0:1] + 1.0
    return lax.rsqrt(deg)


def _tc_embed(x_ref, w1_ref, degp_ref, hn_ref):
    dinv = _dinv_block(degp_ref)
    h1 = jnp.dot(x_ref[:], w1_ref[:], preferred_element_type=jnp.float32)
    hn_ref[:] = h1 * dinv


def _tc_mid(p_ref, hn_ref, degp_ref, out_ref):
    dinv = _dinv_block(degp_ref)
    agg = p_ref[0] + p_ref[1] + hn_ref[:]
    hidden = jnp.maximum(agg * dinv, 0.0)
    out_ref[:] = hidden * dinv


def _tc_latent(p_ref, hn2_ref, degp_ref, w2_ref, w3_ref, mu_ref, lv_ref):
    dinv = _dinv_block(degp_ref)
    sagg = (p_ref[0] + p_ref[1] + hn2_ref[:]) * dinv
    mu_ref[:] = jnp.dot(sagg, w2_ref[:], preferred_element_type=jnp.float32)
    lv_ref[:] = jnp.dot(sagg, w3_ref[:], preferred_element_type=jnp.float32)


BMD = 512
BND = 2048


def _tc_decoder(zi_ref, zj_ref, out_ref):
    prod = lax.dot_general(zi_ref[:], zj_ref[:], (((1,), (1,)), ((), ())),
                           preferred_element_type=jnp.float32)
    out_ref[:] = 0.5 * jnp.tanh(0.5 * prod) + 0.5


def kernel(features, edge_index, W1, W2, W3):
    edges = edge_index.reshape(2, NW, NCHUNK, CHUNK)

    degp = _sc_degree(edges)

    hn = pl.pallas_call(
        _tc_embed,
        grid=(N // BM,),
        in_specs=[
            pl.BlockSpec((BM, D_IN), lambda i: (i, 0)),
            pl.BlockSpec((D_IN, D_HID), lambda i: (0, 0)),
            pl.BlockSpec((NC, BM, 16), lambda i: (0, i, 0)),
        ],
        out_specs=pl.BlockSpec((BM, D_HID), lambda i: (i, 0)),
        out_shape=jax.ShapeDtypeStruct((N, D_HID), jnp.float32),
    )(features, W1, degp)

    p1 = _sc_aggregate(hn, edges)

    hn2 = pl.pallas_call(
        _tc_mid,
        grid=(N // BM,),
        in_specs=[
            pl.BlockSpec((NC, BM, D_HID), lambda i: (0, i, 0)),
            pl.BlockSpec((BM, D_HID), lambda i: (i, 0)),
            pl.BlockSpec((NC, BM, 16), lambda i: (0, i, 0)),
        ],
        out_specs=pl.BlockSpec((BM, D_HID), lambda i: (i, 0)),
        out_shape=jax.ShapeDtypeStruct((N, D_HID), jnp.float32),
    )(p1, hn, degp)

    p2 = _sc_aggregate(hn2, edges)

    mu, logvar = pl.pallas_call(
        _tc_latent,
        grid=(N // BM,),
        in_specs=[
            pl.BlockSpec((NC, BM, D_HID), lambda i: (0, i, 0)),
            pl.BlockSpec((BM, D_HID), lambda i: (i, 0)),
            pl.BlockSpec((NC, BM, 16), lambda i: (0, i, 0)),
            pl.BlockSpec((D_HID, D_LAT), lambda i: (0, 0)),
            pl.BlockSpec((D_HID, D_LAT), lambda i: (0, 0)),
        ],
        out_specs=[
            pl.BlockSpec((BM, D_LAT), lambda i: (i, 0)),
            pl.BlockSpec((BM, D_LAT), lambda i: (i, 0)),
        ],
        out_shape=[
            jax.ShapeDtypeStruct((N, D_LAT), jnp.float32),
            jax.ShapeDtypeStruct((N, D_LAT), jnp.float32),
        ],
    )(p2, hn2, degp, W2, W3)

    adj = pl.pallas_call(
        _tc_decoder,
        grid=(pl.cdiv(N, BMD), pl.cdiv(N, BND)),
        in_specs=[
            pl.BlockSpec((BMD, D_LAT), lambda i, j: (i, 0)),
            pl.BlockSpec((BND, D_LAT), lambda i, j: (j, 0)),
        ],
        out_specs=pl.BlockSpec((BMD, BND), lambda i, j: (i, j)),
        out_shape=jax.ShapeDtypeStruct((N, N), jnp.float32),
    )(mu, mu)

    return adj, mu, logvar


# speedup vs baseline: 10.9291x; 1.1335x over previous
import functools

import jax
import jax.numpy as jnp
from jax import lax
from jax.experimental import pallas as pl
from jax.experimental.pallas import tpu as pltpu
from jax.experimental.pallas import tpu_sc as plsc

N = 10000
E = 320000
D_IN = 128
D_HID = 64
D_LAT = 32

NC = 2
NS = 16
NW = NC * NS
EPW = E // NW
CHUNK = 125
NCHUNK = EPW // CHUNK
RING = 5
NGROUP = NCHUNK // RING
NP = 10240
RPS = NP // NS

_mesh = plsc.VectorSubcoreMesh(core_axis_name="c", subcore_axis_name="s")


_SC_PARAMS = pltpu.CompilerParams(use_tc_tiling_on_sc=False)


@functools.partial(
    pl.kernel,
    out_type=jax.ShapeDtypeStruct((NC, NP, 16), jnp.float32),
    mesh=_mesh,
    compiler_params=_SC_PARAMS,
    scratch_types=[
        pltpu.VMEM((NCHUNK, CHUNK), jnp.int32),
        pltpu.VMEM((CHUNK, 16), jnp.float32),
        pltpu.VMEM((RPS, 16), jnp.float32),
        pltpu.VMEM_SHARED((NP, 16), jnp.float32),
        pltpu.SemaphoreType.DMA,
    ],
)
def _sc_degree(edges_hbm, out_hbm, dst_v, ones_v, zero_v, acc, sem):
    c = lax.axis_index("c")
    s = lax.axis_index("s")
    wid = c * NS + s

    def zfill(i, carry):
        zero_v[i, :] = jnp.zeros((16,), jnp.float32)
        return carry

    lax.fori_loop(0, RPS, zfill, 0)

    def ofill(i, carry):
        ones_v[i, :] = jnp.full((16,), 1.0, jnp.float32)
        return carry

    lax.fori_loop(0, CHUNK, ofill, 0)

    pltpu.sync_copy(zero_v, acc.at[pl.ds(s * RPS, RPS)])
    pltpu.sync_copy(edges_hbm.at[1, wid], dst_v)
    plsc.subcore_barrier()

    def group(o, carry):
        for b in range(RING):
            pltpu.async_copy(ones_v, acc.at[dst_v.at[o * RING + b]], sem,
                             add=True)
        for b in range(RING):
            pltpu.make_async_copy(ones_v, acc.at[dst_v.at[o * RING + b]],
                                  sem).wait()
        return carry

    lax.fori_loop(0, NCHUNK // RING, group, 0)

    plsc.subcore_barrier()
    pltpu.sync_copy(acc.at[pl.ds(s * RPS, RPS)],
                    out_hbm.at[c].at[pl.ds(s * RPS, RPS)])


@functools.partial(
    pl.kernel,
    out_type=jax.ShapeDtypeStruct((NC, NP, D_HID), jnp.float32),
    mesh=_mesh,
    compiler_params=_SC_PARAMS,
    scratch_types=[
        pltpu.VMEM((NCHUNK, CHUNK), jnp.int32),
        pltpu.VMEM((NCHUNK, CHUNK), jnp.int32),
        pltpu.VMEM((RING, CHUNK, D_HID), jnp.float32),
        pltpu.VMEM((RPS // 4, D_HID), jnp.float32),
        pltpu.VMEM_SHARED((NP, D_HID), jnp.float32),
    ] + [pltpu.SemaphoreType.DMA] * (2 * RING),
)
def _sc_aggregate(feat_hbm, edges_hbm, out_hbm,
                  src_v, dst_v, rows_v, zero_v, acc, *sems):
    gsem = sems[:RING]
    ssem = sems[RING:]
    c = lax.axis_index("c")
    s = lax.axis_index("s")
    wid = c * NS + s

    def zfill(i, carry):
        for j in range(D_HID // 16):
            zero_v[i, pl.ds(j * 16, 16)] = jnp.zeros((16,), jnp.float32)
        return carry

    lax.fori_loop(0, RPS // 4, zfill, 0)

    for q in range(4):
        pltpu.sync_copy(zero_v, acc.at[pl.ds(s * RPS + q * (RPS // 4), RPS // 4)])
    pltpu.sync_copy(edges_hbm.at[0, wid], src_v)
    pltpu.sync_copy(edges_hbm.at[1, wid], dst_v)
    plsc.subcore_barrier()

    for b in range(RING):
        pltpu.async_copy(feat_hbm.at[src_v.at[b]], rows_v.at[b], gsem[b])

    def group(o, carry):
        descs = []
        for b in range(RING):
            k = o * RING + b
            pltpu.make_async_copy(feat_hbm.at[src_v.at[k]],
                                  rows_v.at[b], gsem[b]).wait()
            descs.append(pltpu.async_copy(rows_v.at[b],
                                          acc.at[dst_v.at[k]], ssem[b],
                                          add=True))

        @pl.when(o < NGROUP - 1)
        def _refill():
            for b in range(RING):
                descs[b].wait()
                k2 = (o + 1) * RING + b
                pltpu.async_copy(feat_hbm.at[src_v.at[k2]],
                                 rows_v.at[b], gsem[b])

        return carry

    lax.fori_loop(0, NGROUP, group, 0)

    for b in range(RING):
        kf = (NGROUP - 1) * RING + b
        pltpu.make_async_copy(rows_v.at[b], acc.at[dst_v.at[kf]],
                              ssem[b]).wait()

    plsc.subcore_barrier()
    pltpu.sync_copy(acc.at[pl.ds(s * RPS, RPS)],
                    out_hbm.at[c].at[pl.ds(s * RPS, RPS)])


BM = 1000


def _dinv_block(degp_ref):
    deg = degp_ref[0, :, 0:1] + degp_ref[1, :, 0:1] + 1.0
    return lax.rsqrt(deg)


def _tc_embed(x_ref, w1_ref, degp_ref, hn_ref):
    dinv = _dinv_block(degp_ref)
    h1 = jnp.dot(x_ref[:], w1_ref[:], preferred_element_type=jnp.float32)
    hn_ref[:] = h1 * dinv


def _tc_mid(p_ref, hn_ref, degp_ref, out_ref):
    dinv = _dinv_block(degp_ref)
    agg = p_ref[0] + p_ref[1] + hn_ref[:]
    hidden = jnp.maximum(agg * dinv, 0.0)
    out_ref[:] = hidden * dinv


def _tc_latent(p_ref, hn2_ref, degp_ref, w2_ref, w3_ref, mu_ref, lv_ref):
    dinv = _dinv_block(degp_ref)
    sagg = (p_ref[0] + p_ref[1] + hn2_ref[:]) * dinv
    mu_ref[:] = jnp.dot(sagg, w2_ref[:], preferred_element_type=jnp.float32)
    lv_ref[:] = jnp.dot(sagg, w3_ref[:], preferred_element_type=jnp.float32)


BMD = 128


def _tc_decoder(zi_ref, zj_ref, out_ref):
    prod = lax.dot_general(zi_ref[:], zj_ref[:], (((1,), (1,)), ((), ())),
                           preferred_element_type=jnp.float32)
    out_ref[:] = 0.5 * jnp.tanh(0.5 * prod) + 0.5


def kernel(features, edge_index, W1, W2, W3):
    edges = edge_index.reshape(2, NW, NCHUNK, CHUNK)

    degp = _sc_degree(edges)

    hn = pl.pallas_call(
        _tc_embed,
        grid=(N // BM,),
        in_specs=[
            pl.BlockSpec((BM, D_IN), lambda i: (i, 0)),
            pl.BlockSpec((D_IN, D_HID), lambda i: (0, 0)),
            pl.BlockSpec((NC, BM, 16), lambda i: (0, i, 0)),
        ],
        out_specs=pl.BlockSpec((BM, D_HID), lambda i: (i, 0)),
        out_shape=jax.ShapeDtypeStruct((N, D_HID), jnp.float32),
    )(features, W1, degp)

    p1 = _sc_aggregate(hn, edges)

    hn2 = pl.pallas_call(
        _tc_mid,
        grid=(N // BM,),
        in_specs=[
            pl.BlockSpec((NC, BM, D_HID), lambda i: (0, i, 0)),
            pl.BlockSpec((BM, D_HID), lambda i: (i, 0)),
            pl.BlockSpec((NC, BM, 16), lambda i: (0, i, 0)),
        ],
        out_specs=pl.BlockSpec((BM, D_HID), lambda i: (i, 0)),
        out_shape=jax.ShapeDtypeStruct((N, D_HID), jnp.float32),
    )(p1, hn, degp)

    p2 = _sc_aggregate(hn2, edges)

    mu, logvar = pl.pallas_call(
        _tc_latent,
        grid=(N // BM,),
        in_specs=[
            pl.BlockSpec((NC, BM, D_HID), lambda i: (0, i, 0)),
            pl.BlockSpec((BM, D_HID), lambda i: (i, 0)),
            pl.BlockSpec((NC, BM, 16), lambda i: (0, i, 0)),
            pl.BlockSpec((D_HID, D_LAT), lambda i: (0, 0)),
            pl.BlockSpec((D_HID, D_LAT), lambda i: (0, 0)),
        ],
        out_specs=[
            pl.BlockSpec((BM, D_LAT), lambda i: (i, 0)),
            pl.BlockSpec((BM, D_LAT), lambda i: (i, 0)),
        ],
        out_shape=[
            jax.ShapeDtypeStruct((N, D_LAT), jnp.float32),
            jax.ShapeDtypeStruct((N, D_LAT), jnp.float32),
        ],
    )(p2, hn2, degp, W2, W3)

    adj = pl.pallas_call(
        _tc_decoder,
        grid=(pl.cdiv(N, BMD),),
        in_specs=[
            pl.BlockSpec((BMD, D_LAT), lambda i: (i, 0)),
            pl.BlockSpec((N, D_LAT), lambda i: (0, 0)),
        ],
        out_specs=pl.BlockSpec((BMD, N), lambda i: (i, 0)),
        out_shape=jax.ShapeDtypeStruct((N, N), jnp.float32),
    )(mu, mu)

    return adj, mu, logvar
